# baseline (speedup 1.0000x reference)
import jax
import jax.numpy as jnp
from jax.experimental import pallas as pl
from jax.experimental.pallas import tpu as pltpu

N = 16384
C = 1000
K = 4096
BLK = 256
GRID = N // BLK


def _ohem_body(pred_ref, tgt_ref, out_ref, loss_acc):
    i = pl.program_id(0)
    x = pred_ref[...]
    m = jnp.max(x, axis=1, keepdims=True)
    s = jnp.sum(jnp.exp(x - m), axis=1)
    lse = m[:, 0] + jnp.log(s)
    tgt = tgt_ref[0, 0, :]
    col = jax.lax.broadcasted_iota(jnp.int32, (BLK, C), 1)
    tl = jnp.sum(jnp.where(col == tgt[:, None], x, 0.0), axis=1)
    ce = jnp.where(tgt == -1, 0.0, lse - tl)
    loss_acc[pl.ds(i, 1), :] = ce[None, :]

    @pl.when(i == GRID - 1)
    def _select():
        vals = loss_acc[...]
        bits = jax.lax.bitcast_convert_type(vals, jnp.int32)

        def body(j, t):
            cand = t | jax.lax.shift_left(jnp.int32(1), jnp.int32(30) - j)
            c = jnp.sum(jnp.where(bits >= cand, 1, 0))
            return jnp.where(c >= K, cand, t)

        t = jax.lax.fori_loop(0, 31, body, jnp.int32(0))
        gt = bits > t
        cnt_gt = jnp.sum(jnp.where(gt, 1, 0))
        sum_gt = jnp.sum(jnp.where(gt, vals, 0.0))
        thr = jax.lax.bitcast_convert_type(t, jnp.float32)
        total = sum_gt + (jnp.int32(K) - cnt_gt).astype(jnp.float32) * thr
        out_ref[0, 0] = total / jnp.float32(K)


def kernel(pred, target, epoch):
    tgt3 = target.reshape(GRID, 1, BLK)
    out = pl.pallas_call(
        _ohem_body,
        grid=(GRID,),
        in_specs=[
            pl.BlockSpec((BLK, C), lambda i: (i, 0)),
            pl.BlockSpec((1, 1, BLK), lambda i: (i, 0, 0)),
        ],
        out_specs=pl.BlockSpec(memory_space=pltpu.SMEM),
        out_shape=jax.ShapeDtypeStruct((1, 1), jnp.float32),
        scratch_shapes=[pltpu.VMEM((GRID, BLK), jnp.float32)],
    )(pred, tgt3)
    return out[0, 0]
